# Initial kernel scaffold; baseline (speedup 1.0000x reference)
import functools

import jax
import jax.numpy as jnp
from jax import lax
from jax.experimental import pallas as pl
from jax.experimental.pallas import tpu as pltpu
from jax.experimental.pallas import tpu_sc as plsc

_N = 10000
_E = 320000
_D = 128
_C = 40
_L = 3
_EPS = 1e-5

_NC = 2
_NS = 16
_NW = _NC * _NS
_EPW = _E // _NW
_K = 80
_NCHUNK = _EPW // _K
_RPT = _N // _NS
_ZROWS = 125

_mesh = plsc.VectorSubcoreMesh(core_axis_name="c", subcore_axis_name="s")


@functools.partial(
    pl.kernel,
    out_type=jax.ShapeDtypeStruct((_NC, _N, 16), jnp.float32),
    mesh=_mesh,
    scratch_types=[
        pltpu.VMEM((_K,), jnp.int32),
        pltpu.VMEM((_K, 16), jnp.float32),
        pltpu.VMEM((_RPT, 16), jnp.float32),
        pltpu.VMEM_SHARED((_N, 16), jnp.float32),
    ],
)
def _sc_degree(col_hbm, out_hbm, idx_v, ones_v, zb_v, acc_sh):
    c = lax.axis_index("c")
    s = lax.axis_index("s")
    w = c * _NS + s

    def _fill_zero(i, carry):
        zb_v[i, :] = jnp.zeros((16,), jnp.float32)
        return carry

    lax.fori_loop(0, _RPT, _fill_zero, 0)

    def _fill_one(i, carry):
        ones_v[i, :] = jnp.ones((16,), jnp.float32)
        return carry

    lax.fori_loop(0, _K, _fill_one, 0)

    pltpu.sync_copy(zb_v, acc_sh.at[pl.ds(s * _RPT, _RPT)])
    plsc.subcore_barrier()

    def _step(i, carry):
        base = w * _EPW + i * _K
        pltpu.sync_copy(col_hbm.at[pl.ds(base, _K)], idx_v)
        pltpu.sync_copy(ones_v, acc_sh.at[idx_v], add=True)
        return carry

    lax.fori_loop(0, _NCHUNK, _step, 0)
    plsc.subcore_barrier()
    pltpu.sync_copy(
        acc_sh.at[pl.ds(s * _RPT, _RPT)], out_hbm.at[c, pl.ds(s * _RPT, _RPT)]
    )


@functools.partial(
    pl.kernel,
    out_type=jax.ShapeDtypeStruct((_NC, _N, _D), jnp.float32),
    mesh=_mesh,
    scratch_types=[
        pltpu.VMEM((_K,), jnp.int32),
        pltpu.VMEM((_K,), jnp.int32),
        pltpu.VMEM((_K, _D), jnp.float32),
        pltpu.VMEM((_ZROWS, _D), jnp.float32),
        pltpu.VMEM_SHARED((_N, _D), jnp.float32),
        pltpu.SemaphoreType.DMA,
    ],
)
def _sc_aggregate(hp_hbm, row_hbm, col_hbm, out_hbm, ri_v, ci_v, rows_v, zb_v,
                  acc_sh, sem):
    c = lax.axis_index("c")
    s = lax.axis_index("s")
    w = c * _NS + s

    def _fill_zero(i, carry):
        for j in range(_D // 16):
            zb_v[i, pl.ds(j * 16, 16)] = jnp.zeros((16,), jnp.float32)
        return carry

    lax.fori_loop(0, _ZROWS, _fill_zero, 0)

    def _zero_acc(k, carry):
        pltpu.sync_copy(zb_v, acc_sh.at[pl.ds(s * _RPT + k * _ZROWS, _ZROWS)])
        return carry

    lax.fori_loop(0, _RPT // _ZROWS, _zero_acc, 0)
    plsc.subcore_barrier()

    def _step(i, carry):
        base = w * _EPW + i * _K
        pltpu.sync_copy(row_hbm.at[pl.ds(base, _K)], ri_v)
        pltpu.sync_copy(col_hbm.at[pl.ds(base, _K)], ci_v)
        pltpu.async_copy(hp_hbm.at[ri_v], rows_v, sem).wait()
        pltpu.sync_copy(rows_v, acc_sh.at[ci_v], add=True)
        return carry

    lax.fori_loop(0, _NCHUNK, _step, 0)
    plsc.subcore_barrier()
    pltpu.sync_copy(
        acc_sh.at[pl.ds(s * _RPT, _RPT)], out_hbm.at[c, pl.ds(s * _RPT, _RPT)]
    )


_BN_ROWS = 1000
_GRID = _N // _BN_ROWS


def _dinv_body(p_ref, o_ref):
    d = p_ref[0] + p_ref[1]
    dv = jnp.where(d > 0, lax.rsqrt(d), 0.0)
    o_ref[...] = jnp.broadcast_to(dv[:, 0:1], (_N, _D))


def _dinv_call(parts):
    return pl.pallas_call(
        _dinv_body,
        out_shape=jax.ShapeDtypeStruct((_N, _D), jnp.float32),
    )(parts)


def _enc_body(x_ref, w_ref, b_ref, o_ref):
    h = lax.dot_general(x_ref[...], w_ref[...], (((1,), (1,)), ((), ())),
                        preferred_element_type=jnp.float32)
    o_ref[...] = jnp.maximum(h + b_ref[...], 0.0)


def _enc_call(x, w, b):
    return pl.pallas_call(
        _enc_body,
        grid=(_GRID,),
        in_specs=[
            pl.BlockSpec((_BN_ROWS, _D), lambda i: (i, 0)),
            pl.BlockSpec((_D, _D), lambda i: (0, 0)),
            pl.BlockSpec((1, _D), lambda i: (0, 0)),
        ],
        out_specs=pl.BlockSpec((_BN_ROWS, _D), lambda i: (i, 0)),
        out_shape=jax.ShapeDtypeStruct((_N, _D), jnp.float32),
    )(x, w, b)


def _smm_body(h_ref, w_ref, dv_ref, o_ref):
    hw = lax.dot_general(h_ref[...], w_ref[...], (((1,), (1,)), ((), ())),
                         preferred_element_type=jnp.float32)
    o_ref[...] = hw * dv_ref[...]


def _smm_call(h, w, dinv):
    return pl.pallas_call(
        _smm_body,
        grid=(_GRID,),
        in_specs=[
            pl.BlockSpec((_BN_ROWS, _D), lambda i: (i, 0)),
            pl.BlockSpec((_D, _D), lambda i: (0, 0)),
            pl.BlockSpec((_BN_ROWS, _D), lambda i: (i, 0)),
        ],
        out_specs=pl.BlockSpec((_BN_ROWS, _D), lambda i: (i, 0)),
        out_shape=jax.ShapeDtypeStruct((_N, _D), jnp.float32),
    )(h, w, dinv)


def _combine_body(p_ref, dv_ref, b_ref, y_ref, s_ref, ss_ref):
    i = pl.program_id(0)
    y = (p_ref[0] + p_ref[1]) * dv_ref[...] + b_ref[...]
    y_ref[...] = y

    @pl.when(i == 0)
    def _():
        s_ref[...] = jnp.zeros_like(s_ref)
        ss_ref[...] = jnp.zeros_like(ss_ref)

    s_ref[...] += jnp.sum(y, axis=0, keepdims=True)
    ss_ref[...] += jnp.sum(y * y, axis=0, keepdims=True)


def _combine_res_body(p_ref, dv_ref, b_ref, r_ref, y_ref, s_ref, ss_ref):
    i = pl.program_id(0)
    y = (p_ref[0] + p_ref[1]) * dv_ref[...] + b_ref[...] + r_ref[...]
    y_ref[...] = y

    @pl.when(i == 0)
    def _():
        s_ref[...] = jnp.zeros_like(s_ref)
        ss_ref[...] = jnp.zeros_like(ss_ref)

    s_ref[...] += jnp.sum(y, axis=0, keepdims=True)
    ss_ref[...] += jnp.sum(y * y, axis=0, keepdims=True)


def _combine_call(parts, dinv, b, res=None):
    out_shape = [
        jax.ShapeDtypeStruct((_N, _D), jnp.float32),
        jax.ShapeDtypeStruct((1, _D), jnp.float32),
        jax.ShapeDtypeStruct((1, _D), jnp.float32),
    ]
    in_specs = [
        pl.BlockSpec((_NC, _BN_ROWS, _D), lambda i: (0, i, 0)),
        pl.BlockSpec((_BN_ROWS, _D), lambda i: (i, 0)),
        pl.BlockSpec((1, _D), lambda i: (0, 0)),
    ]
    args = [parts, dinv, b]
    body = _combine_body
    if res is not None:
        in_specs.append(pl.BlockSpec((_BN_ROWS, _D), lambda i: (i, 0)))
        args.append(res)
        body = _combine_res_body
    return pl.pallas_call(
        body,
        grid=(_GRID,),
        in_specs=in_specs,
        out_specs=[
            pl.BlockSpec((_BN_ROWS, _D), lambda i: (i, 0)),
            pl.BlockSpec((1, _D), lambda i: (0, 0)),
            pl.BlockSpec((1, _D), lambda i: (0, 0)),
        ],
        out_shape=out_shape,
    )(*args)


def _apply_mm_body(y_ref, s_ref, ss_ref, g_ref, be_ref, w_ref, dv_ref, o_ref):
    mu = s_ref[...] / _N
    var = ss_ref[...] / _N - mu * mu
    inv = lax.rsqrt(var + _EPS)
    h = jnp.maximum((y_ref[...] - mu) * inv * g_ref[...] + be_ref[...], 0.0)
    hw = lax.dot_general(h, w_ref[...], (((1,), (1,)), ((), ())),
                         preferred_element_type=jnp.float32)
    o_ref[...] = hw * dv_ref[...]


def _apply_mm_call(y, s_, ss_, g, be, w, dinv):
    return pl.pallas_call(
        _apply_mm_body,
        grid=(_GRID,),
        in_specs=[
            pl.BlockSpec((_BN_ROWS, _D), lambda i: (i, 0)),
            pl.BlockSpec((1, _D), lambda i: (0, 0)),
            pl.BlockSpec((1, _D), lambda i: (0, 0)),
            pl.BlockSpec((1, _D), lambda i: (0, 0)),
            pl.BlockSpec((1, _D), lambda i: (0, 0)),
            pl.BlockSpec((_D, _D), lambda i: (0, 0)),
            pl.BlockSpec((_BN_ROWS, _D), lambda i: (i, 0)),
        ],
        out_specs=pl.BlockSpec((_BN_ROWS, _D), lambda i: (i, 0)),
        out_shape=jax.ShapeDtypeStruct((_N, _D), jnp.float32),
    )(y, s_, ss_, g, be, w, dinv)


def _apply_pred_body(y_ref, s_ref, ss_ref, g_ref, be_ref, w_ref, bp_ref, o_ref):
    mu = s_ref[...] / _N
    var = ss_ref[...] / _N - mu * mu
    inv = lax.rsqrt(var + _EPS)
    h = jnp.maximum((y_ref[...] - mu) * inv * g_ref[...] + be_ref[...], 0.0)
    hw = lax.dot_general(h, w_ref[...], (((1,), (1,)), ((), ())),
                         preferred_element_type=jnp.float32)
    o_ref[...] = hw + bp_ref[...]


def _apply_pred_call(y, s_, ss_, g, be, wp, bp):
    return pl.pallas_call(
        _apply_pred_body,
        grid=(_GRID,),
        in_specs=[
            pl.BlockSpec((_BN_ROWS, _D), lambda i: (i, 0)),
            pl.BlockSpec((1, _D), lambda i: (0, 0)),
            pl.BlockSpec((1, _D), lambda i: (0, 0)),
            pl.BlockSpec((1, _D), lambda i: (0, 0)),
            pl.BlockSpec((1, _D), lambda i: (0, 0)),
            pl.BlockSpec((_D, _D), lambda i: (0, 0)),
            pl.BlockSpec((1, _D), lambda i: (0, 0)),
        ],
        out_specs=pl.BlockSpec((_BN_ROWS, _D), lambda i: (i, 0)),
        out_shape=jax.ShapeDtypeStruct((_N, _D), jnp.float32),
    )(y, s_, ss_, g, be, wp, bp)


def kernel(x, edge_index, W_enc, b_enc, Ws, bs, gammas, betas, W_pred, b_pred):
    row = edge_index[0]
    col = edge_index[1]

    deg_parts = _sc_degree(col)
    dinv = _dinv_call(deg_parts)

    h = _enc_call(x, W_enc, b_enc.reshape(1, _D))
    hp = _smm_call(h, Ws[0], dinv)

    wp_pad = jnp.zeros((_D, _D), jnp.float32).at[:_C].set(W_pred)
    bp_pad = jnp.zeros((1, _D), jnp.float32).at[0, :_C].set(b_pred)

    y_last = None
    out = None
    for i in range(_L):
        parts = _sc_aggregate(hp, row, col)
        y, s_, ss_ = _combine_call(parts, dinv, bs[i].reshape(1, _D), y_last)
        y_last = y
        g = gammas[i].reshape(1, _D)
        be = betas[i].reshape(1, _D)
        if i < _L - 1:
            hp = _apply_mm_call(y, s_, ss_, g, be, Ws[i + 1], dinv)
        else:
            out = _apply_pred_call(y, s_, ss_, g, be, wp_pad, bp_pad)
    return out[:, :_C]


# speedup vs baseline: 8.4397x; 8.4397x over previous
import functools

import jax
import jax.numpy as jnp
from jax import lax
from jax.experimental import pallas as pl
from jax.experimental.pallas import tpu as pltpu
from jax.experimental.pallas import tpu_sc as plsc

_N = 10000
_E = 320000
_D = 128
_C = 40
_L = 3
_EPS = 1e-5

_NC = 2
_NS = 16
_NW = _NC * _NS
_EPW = _E // _NW
_K = 80
_NCHUNK = _EPW // _K
_NP = 10240
_RPT = _NP // _NS
_ZROWS = 128

_mesh = plsc.VectorSubcoreMesh(core_axis_name="c", subcore_axis_name="s")


@functools.partial(
    pl.kernel,
    out_type=jax.ShapeDtypeStruct((_NC, _NP, _D), jnp.float32),
    mesh=_mesh,
    scratch_types=[
        pltpu.VMEM((_K,), jnp.int32),
        pltpu.VMEM((_K, _D), jnp.float32),
        pltpu.VMEM((_ZROWS, _D), jnp.float32),
        pltpu.VMEM_SHARED((_NP, _D), jnp.float32),
    ],
)
def _sc_degree(col_hbm, out_hbm, ci_v, ones_v, zb_v, acc_sh):
    c = lax.axis_index("c")
    s = lax.axis_index("s")
    w = c * _NS + s

    def _fill_zero(i, carry):
        for j in range(_D // 16):
            zb_v[i, pl.ds(j * 16, 16)] = jnp.zeros((16,), jnp.float32)
        return carry

    lax.fori_loop(0, _ZROWS, _fill_zero, 0)

    def _fill_one(i, carry):
        for j in range(_D // 16):
            ones_v[i, pl.ds(j * 16, 16)] = jnp.ones((16,), jnp.float32)
        return carry

    lax.fori_loop(0, _K, _fill_one, 0)

    def _zero_acc(k, carry):
        pltpu.sync_copy(zb_v, acc_sh.at[pl.ds(s * _RPT + k * _ZROWS, _ZROWS)])
        return carry

    lax.fori_loop(0, _RPT // _ZROWS, _zero_acc, 0)
    plsc.subcore_barrier()

    def _step(i, carry):
        base = w * _EPW + i * _K
        pltpu.sync_copy(col_hbm.at[pl.ds(base, _K)], ci_v)
        pltpu.sync_copy(ones_v, acc_sh.at[ci_v], add=True)
        return carry

    lax.fori_loop(0, _NCHUNK, _step, 0)
    plsc.subcore_barrier()
    pltpu.sync_copy(
        acc_sh.at[pl.ds(s * _RPT, _RPT)], out_hbm.at[c, pl.ds(s * _RPT, _RPT)]
    )


@functools.partial(
    pl.kernel,
    out_type=jax.ShapeDtypeStruct((_NC, _NP, _D), jnp.float32),
    mesh=_mesh,
    scratch_types=[
        pltpu.VMEM((_K,), jnp.int32),
        pltpu.VMEM((_K,), jnp.int32),
        pltpu.VMEM((_K, _D), jnp.float32),
        pltpu.VMEM((_ZROWS, _D), jnp.float32),
        pltpu.VMEM_SHARED((_NP, _D), jnp.float32),
        pltpu.SemaphoreType.DMA,
    ],
)
def _sc_aggregate(hp_hbm, row_hbm, col_hbm, out_hbm, ri_v, ci_v, rows_v, zb_v,
                  acc_sh, sem):
    c = lax.axis_index("c")
    s = lax.axis_index("s")
    w = c * _NS + s

    def _fill_zero(i, carry):
        for j in range(_D // 16):
            zb_v[i, pl.ds(j * 16, 16)] = jnp.zeros((16,), jnp.float32)
        return carry

    lax.fori_loop(0, _ZROWS, _fill_zero, 0)

    def _zero_acc(k, carry):
        pltpu.sync_copy(zb_v, acc_sh.at[pl.ds(s * _RPT + k * _ZROWS, _ZROWS)])
        return carry

    lax.fori_loop(0, _RPT // _ZROWS, _zero_acc, 0)
    plsc.subcore_barrier()

    def _step(i, carry):
        base = w * _EPW + i * _K
        pltpu.sync_copy(row_hbm.at[pl.ds(base, _K)], ri_v)
        pltpu.sync_copy(col_hbm.at[pl.ds(base, _K)], ci_v)
        pltpu.async_copy(hp_hbm.at[ri_v], rows_v, sem).wait()
        pltpu.sync_copy(rows_v, acc_sh.at[ci_v], add=True)
        return carry

    lax.fori_loop(0, _NCHUNK, _step, 0)
    plsc.subcore_barrier()
    pltpu.sync_copy(
        acc_sh.at[pl.ds(s * _RPT, _RPT)], out_hbm.at[c, pl.ds(s * _RPT, _RPT)]
    )


_BN_ROWS = 1000
_GRID = _N // _BN_ROWS


def _dinv_body(p_ref, o_ref):
    d = p_ref[0, :_N] + p_ref[1, :_N]
    o_ref[...] = jnp.where(d > 0, lax.rsqrt(d), 0.0)


def _dinv_call(parts):
    return pl.pallas_call(
        _dinv_body,
        out_shape=jax.ShapeDtypeStruct((_N, _D), jnp.float32),
    )(parts)


def _enc_body(x_ref, w_ref, b_ref, o_ref):
    h = lax.dot_general(x_ref[...], w_ref[...], (((1,), (1,)), ((), ())),
                        preferred_element_type=jnp.float32)
    o_ref[...] = jnp.maximum(h + b_ref[...], 0.0)


def _enc_call(x, w, b):
    return pl.pallas_call(
        _enc_body,
        grid=(_GRID,),
        in_specs=[
            pl.BlockSpec((_BN_ROWS, _D), lambda i: (i, 0)),
            pl.BlockSpec((_D, _D), lambda i: (0, 0)),
            pl.BlockSpec((1, _D), lambda i: (0, 0)),
        ],
        out_specs=pl.BlockSpec((_BN_ROWS, _D), lambda i: (i, 0)),
        out_shape=jax.ShapeDtypeStruct((_N, _D), jnp.float32),
    )(x, w, b)


def _smm_body(h_ref, w_ref, dv_ref, o_ref):
    hw = lax.dot_general(h_ref[...], w_ref[...], (((1,), (1,)), ((), ())),
                         preferred_element_type=jnp.float32)
    o_ref[...] = hw * dv_ref[...]


def _smm_call(h, w, dinv):
    return pl.pallas_call(
        _smm_body,
        grid=(_GRID,),
        in_specs=[
            pl.BlockSpec((_BN_ROWS, _D), lambda i: (i, 0)),
            pl.BlockSpec((_D, _D), lambda i: (0, 0)),
            pl.BlockSpec((_BN_ROWS, _D), lambda i: (i, 0)),
        ],
        out_specs=pl.BlockSpec((_BN_ROWS, _D), lambda i: (i, 0)),
        out_shape=jax.ShapeDtypeStruct((_N, _D), jnp.float32),
    )(h, w, dinv)


def _combine_body(p_ref, dv_ref, b_ref, y_ref, s_ref, ss_ref):
    i = pl.program_id(0)
    y = (p_ref[0] + p_ref[1]) * dv_ref[...] + b_ref[...]
    y_ref[...] = y

    @pl.when(i == 0)
    def _():
        s_ref[...] = jnp.zeros_like(s_ref)
        ss_ref[...] = jnp.zeros_like(ss_ref)

    s_ref[...] += jnp.sum(y, axis=0, keepdims=True)
    ss_ref[...] += jnp.sum(y * y, axis=0, keepdims=True)


def _combine_res_body(p_ref, dv_ref, b_ref, r_ref, y_ref, s_ref, ss_ref):
    i = pl.program_id(0)
    y = (p_ref[0] + p_ref[1]) * dv_ref[...] + b_ref[...] + r_ref[...]
    y_ref[...] = y

    @pl.when(i == 0)
    def _():
        s_ref[...] = jnp.zeros_like(s_ref)
        ss_ref[...] = jnp.zeros_like(ss_ref)

    s_ref[...] += jnp.sum(y, axis=0, keepdims=True)
    ss_ref[...] += jnp.sum(y * y, axis=0, keepdims=True)


def _combine_call(parts, dinv, b, res=None):
    out_shape = [
        jax.ShapeDtypeStruct((_N, _D), jnp.float32),
        jax.ShapeDtypeStruct((1, _D), jnp.float32),
        jax.ShapeDtypeStruct((1, _D), jnp.float32),
    ]
    in_specs = [
        pl.BlockSpec((_NC, _BN_ROWS, _D), lambda i: (0, i, 0)),
        pl.BlockSpec((_BN_ROWS, _D), lambda i: (i, 0)),
        pl.BlockSpec((1, _D), lambda i: (0, 0)),
    ]
    args = [parts, dinv, b]
    body = _combine_body
    if res is not None:
        in_specs.append(pl.BlockSpec((_BN_ROWS, _D), lambda i: (i, 0)))
        args.append(res)
        body = _combine_res_body
    return pl.pallas_call(
        body,
        grid=(_GRID,),
        in_specs=in_specs,
        out_specs=[
            pl.BlockSpec((_BN_ROWS, _D), lambda i: (i, 0)),
            pl.BlockSpec((1, _D), lambda i: (0, 0)),
            pl.BlockSpec((1, _D), lambda i: (0, 0)),
        ],
        out_shape=out_shape,
    )(*args)


def _apply_mm_body(y_ref, s_ref, ss_ref, g_ref, be_ref, w_ref, dv_ref, o_ref):
    mu = s_ref[...] / _N
    var = ss_ref[...] / _N - mu * mu
    inv = lax.rsqrt(var + _EPS)
    h = jnp.maximum((y_ref[...] - mu) * inv * g_ref[...] + be_ref[...], 0.0)
    hw = lax.dot_general(h, w_ref[...], (((1,), (1,)), ((), ())),
                         preferred_element_type=jnp.float32)
    o_ref[...] = hw * dv_ref[...]


def _apply_mm_call(y, s_, ss_, g, be, w, dinv):
    return pl.pallas_call(
        _apply_mm_body,
        grid=(_GRID,),
        in_specs=[
            pl.BlockSpec((_BN_ROWS, _D), lambda i: (i, 0)),
            pl.BlockSpec((1, _D), lambda i: (0, 0)),
            pl.BlockSpec((1, _D), lambda i: (0, 0)),
            pl.BlockSpec((1, _D), lambda i: (0, 0)),
            pl.BlockSpec((1, _D), lambda i: (0, 0)),
            pl.BlockSpec((_D, _D), lambda i: (0, 0)),
            pl.BlockSpec((_BN_ROWS, _D), lambda i: (i, 0)),
        ],
        out_specs=pl.BlockSpec((_BN_ROWS, _D), lambda i: (i, 0)),
        out_shape=jax.ShapeDtypeStruct((_N, _D), jnp.float32),
    )(y, s_, ss_, g, be, w, dinv)


def _apply_pred_body(y_ref, s_ref, ss_ref, g_ref, be_ref, w_ref, bp_ref, o_ref):
    mu = s_ref[...] / _N
    var = ss_ref[...] / _N - mu * mu
    inv = lax.rsqrt(var + _EPS)
    h = jnp.maximum((y_ref[...] - mu) * inv * g_ref[...] + be_ref[...], 0.0)
    hw = lax.dot_general(h, w_ref[...], (((1,), (1,)), ((), ())),
                         preferred_element_type=jnp.float32)
    o_ref[...] = hw + bp_ref[...]


def _apply_pred_call(y, s_, ss_, g, be, wp, bp):
    return pl.pallas_call(
        _apply_pred_body,
        grid=(_GRID,),
        in_specs=[
            pl.BlockSpec((_BN_ROWS, _D), lambda i: (i, 0)),
            pl.BlockSpec((1, _D), lambda i: (0, 0)),
            pl.BlockSpec((1, _D), lambda i: (0, 0)),
            pl.BlockSpec((1, _D), lambda i: (0, 0)),
            pl.BlockSpec((1, _D), lambda i: (0, 0)),
            pl.BlockSpec((_D, _D), lambda i: (0, 0)),
            pl.BlockSpec((1, _D), lambda i: (0, 0)),
        ],
        out_specs=pl.BlockSpec((_BN_ROWS, _D), lambda i: (i, 0)),
        out_shape=jax.ShapeDtypeStruct((_N, _D), jnp.float32),
    )(y, s_, ss_, g, be, wp, bp)


def kernel(x, edge_index, W_enc, b_enc, Ws, bs, gammas, betas, W_pred, b_pred):
    row = edge_index[0]
    col = edge_index[1]

    deg_parts = _sc_degree(col)
    dinv = _dinv_call(deg_parts)

    h = _enc_call(x, W_enc, b_enc.reshape(1, _D))
    hp = _smm_call(h, Ws[0], dinv)

    wp_pad = jnp.zeros((_D, _D), jnp.float32).at[:_C].set(W_pred)
    bp_pad = jnp.zeros((1, _D), jnp.float32).at[0, :_C].set(b_pred)

    y_last = None
    out = None
    for i in range(_L):
        parts = _sc_aggregate(hp, row, col)
        y, s_, ss_ = _combine_call(parts, dinv, bs[i].reshape(1, _D), y_last)
        y_last = y
        g = gammas[i].reshape(1, _D)
        be = betas[i].reshape(1, _D)
        if i < _L - 1:
            hp = _apply_mm_call(y, s_, ss_, g, be, Ws[i + 1], dinv)
        else:
            out = _apply_pred_call(y, s_, ss_, g, be, wp_pad, bp_pad)
    return out[:, :_C]


# speedup vs baseline: 15.8789x; 1.8815x over previous
import functools

import jax
import jax.numpy as jnp
from jax import lax
from jax.experimental import pallas as pl
from jax.experimental.pallas import tpu as pltpu
from jax.experimental.pallas import tpu_sc as plsc

_N = 10000
_E = 320000
_D = 128
_C = 40
_L = 3
_EPS = 1e-5

_NC = 2
_NS = 16
_NW = _NC * _NS
_EPW = _E // _NW
_K = 125
_NCHUNK = _EPW // _K
_NP = 10240
_RPT = _NP // _NS
_ZROWS = 128

_mesh = plsc.VectorSubcoreMesh(core_axis_name="c", subcore_axis_name="s")

_ZC = 80


@functools.partial(
    pl.kernel,
    out_type=jax.ShapeDtypeStruct((_NC, _NP, _D), jnp.float32),
    mesh=_mesh,
    scratch_types=[
        pltpu.VMEM((_NCHUNK, _K), jnp.int32),
        pltpu.VMEM((_K, _D), jnp.float32),
        pltpu.VMEM_SHARED((_NP, _D), jnp.float32),
        pltpu.SemaphoreType.DMA,
    ],
)
def _sc_degree(colw_hbm, dummy_hbm, out_hbm, idx_v, ones_v, acc_sh, sem):
    c = lax.axis_index("c")
    s = lax.axis_index("s")
    w = c * _NS + s

    pltpu.async_copy(colw_hbm.at[w], idx_v, sem)

    def _fill(val):
        def _f(i, carry):
            for j in range(_D // 16):
                ones_v[i, pl.ds(j * 16, 16)] = jnp.full((16,), val, jnp.float32)
            return carry
        return _f

    lax.fori_loop(0, _ZC, _fill(0.0), 0)

    def _zero_acc(k, carry):
        pltpu.sync_copy(ones_v.at[pl.ds(0, _ZC)],
                        acc_sh.at[pl.ds(s * _RPT + k * _ZC, _ZC)])
        return carry

    lax.fori_loop(0, _RPT // _ZC, _zero_acc, 0)
    lax.fori_loop(0, _K, _fill(1.0), 0)
    pltpu.make_async_copy(colw_hbm.at[w], idx_v, sem).wait()
    plsc.subcore_barrier()

    _W = 8

    def _step(i, carry):
        pltpu.async_copy(ones_v, acc_sh.at[idx_v.at[i]], sem, add=True)

        @pl.when(i >= _W)
        def _():
            pltpu.make_async_copy(dummy_hbm, ones_v, sem).wait()

        return carry

    lax.fori_loop(0, _NCHUNK, _step, 0)

    def _drain(i, carry):
        pltpu.make_async_copy(dummy_hbm, ones_v, sem).wait()
        return carry

    lax.fori_loop(0, _W, _drain, 0)
    plsc.subcore_barrier()
    pltpu.sync_copy(
        acc_sh.at[pl.ds(s * _RPT, _RPT)], out_hbm.at[c, pl.ds(s * _RPT, _RPT)]
    )


_HCH = _NCHUNK // 4


@functools.partial(
    pl.kernel,
    out_type=jax.ShapeDtypeStruct((_NC, _NP, _D), jnp.float32),
    mesh=_mesh,
    scratch_types=[
        pltpu.VMEM((2 * _HCH, _K), jnp.int32),
        pltpu.VMEM((_K, _D), jnp.float32),
        pltpu.VMEM((_K, _D), jnp.float32),
        pltpu.VMEM_SHARED((_NP, _D), jnp.float32),
        pltpu.SemaphoreType.DMA,
        pltpu.SemaphoreType.DMA,
        pltpu.SemaphoreType.DMA,
        pltpu.SemaphoreType.DMA,
    ],
)
def _sc_aggregate(hp_hbm, eiw_hbm, dummy_hbm, out_hbm, idx_v, buf0, buf1,
                  acc_sh, sg0, sg1, ss0, ss1):
    c = lax.axis_index("c")
    s = lax.axis_index("s")
    w = c * _NS + s

    def _fz(i, carry):
        for j in range(_D // 16):
            buf0[i, pl.ds(j * 16, 16)] = jnp.zeros((16,), jnp.float32)
        return carry

    lax.fori_loop(0, _ZC, _fz, 0)

    def _zero_acc(k, carry):
        pltpu.sync_copy(buf0.at[pl.ds(0, _ZC)],
                        acc_sh.at[pl.ds(s * _RPT + k * _ZC, _ZC)])
        return carry

    lax.fori_loop(0, _RPT // _ZC, _zero_acc, 0)
    plsc.subcore_barrier()

    def _wait_gather(buf, sg):
        pltpu.make_async_copy(dummy_hbm, buf, sg).wait()

    def _drain_scatter(buf, ss):
        pltpu.make_async_copy(dummy_hbm, buf, ss).wait()

    for h in range(4):
        pltpu.sync_copy(eiw_hbm.at[w, pl.ds(h * 2 * _HCH, 2 * _HCH)], idx_v)

        def _step(g, carry):
            i0 = 2 * g
            i1 = i0 + 1

            @pl.when(g > 0)
            def _():
                _drain_scatter(buf0, ss0)

            pltpu.async_copy(hp_hbm.at[idx_v.at[2 * i0]], buf0, sg0)

            @pl.when(g > 0)
            def _():
                _drain_scatter(buf1, ss1)

            pltpu.async_copy(hp_hbm.at[idx_v.at[2 * i1]], buf1, sg1)
            _wait_gather(buf0, sg0)
            pltpu.async_copy(buf0, acc_sh.at[idx_v.at[2 * i0 + 1]], ss0, add=True)
            _wait_gather(buf1, sg1)
            pltpu.async_copy(buf1, acc_sh.at[idx_v.at[2 * i1 + 1]], ss1, add=True)
            return carry

        lax.fori_loop(0, _HCH // 2, _step, 0)
        _drain_scatter(buf0, ss0)
        _drain_scatter(buf1, ss1)

    plsc.subcore_barrier()
    pltpu.sync_copy(
        acc_sh.at[pl.ds(s * _RPT, _RPT)], out_hbm.at[c, pl.ds(s * _RPT, _RPT)]
    )


_BN_ROWS = 1000
_GRID = _N // _BN_ROWS


def _dinv_body(p_ref, o_ref):
    d = p_ref[0, :_N] + p_ref[1, :_N]
    o_ref[...] = jnp.where(d > 0, lax.rsqrt(d), 0.0)


def _dinv_call(parts):
    return pl.pallas_call(
        _dinv_body,
        out_shape=jax.ShapeDtypeStruct((_N, _D), jnp.float32),
    )(parts)


def _enc_body(x_ref, w_ref, b_ref, o_ref):
    h = lax.dot_general(x_ref[...], w_ref[...], (((1,), (1,)), ((), ())),
                        preferred_element_type=jnp.float32)
    o_ref[...] = jnp.maximum(h + b_ref[...], 0.0)


def _enc_call(x, w, b):
    return pl.pallas_call(
        _enc_body,
        grid=(_GRID,),
        in_specs=[
            pl.BlockSpec((_BN_ROWS, _D), lambda i: (i, 0)),
            pl.BlockSpec((_D, _D), lambda i: (0, 0)),
            pl.BlockSpec((1, _D), lambda i: (0, 0)),
        ],
        out_specs=pl.BlockSpec((_BN_ROWS, _D), lambda i: (i, 0)),
        out_shape=jax.ShapeDtypeStruct((_N, _D), jnp.float32),
    )(x, w, b)


def _smm_body(h_ref, w_ref, dv_ref, o_ref):
    hw = lax.dot_general(h_ref[...], w_ref[...], (((1,), (1,)), ((), ())),
                         preferred_element_type=jnp.float32)
    o_ref[...] = hw * dv_ref[...]


def _smm_call(h, w, dinv):
    return pl.pallas_call(
        _smm_body,
        grid=(_GRID,),
        in_specs=[
            pl.BlockSpec((_BN_ROWS, _D), lambda i: (i, 0)),
            pl.BlockSpec((_D, _D), lambda i: (0, 0)),
            pl.BlockSpec((_BN_ROWS, _D), lambda i: (i, 0)),
        ],
        out_specs=pl.BlockSpec((_BN_ROWS, _D), lambda i: (i, 0)),
        out_shape=jax.ShapeDtypeStruct((_N, _D), jnp.float32),
    )(h, w, dinv)


def _combine_body(p_ref, dv_ref, b_ref, y_ref, s_ref, ss_ref):
    i = pl.program_id(0)
    y = (p_ref[0] + p_ref[1]) * dv_ref[...] + b_ref[...]
    y_ref[...] = y

    @pl.when(i == 0)
    def _():
        s_ref[...] = jnp.zeros_like(s_ref)
        ss_ref[...] = jnp.zeros_like(ss_ref)

    s_ref[...] += jnp.sum(y, axis=0, keepdims=True)
    ss_ref[...] += jnp.sum(y * y, axis=0, keepdims=True)


def _combine_res_body(p_ref, dv_ref, b_ref, r_ref, y_ref, s_ref, ss_ref):
    i = pl.program_id(0)
    y = (p_ref[0] + p_ref[1]) * dv_ref[...] + b_ref[...] + r_ref[...]
    y_ref[...] = y

    @pl.when(i == 0)
    def _():
        s_ref[...] = jnp.zeros_like(s_ref)
        ss_ref[...] = jnp.zeros_like(ss_ref)

    s_ref[...] += jnp.sum(y, axis=0, keepdims=True)
    ss_ref[...] += jnp.sum(y * y, axis=0, keepdims=True)


def _combine_call(parts, dinv, b, res=None):
    out_shape = [
        jax.ShapeDtypeStruct((_N, _D), jnp.float32),
        jax.ShapeDtypeStruct((1, _D), jnp.float32),
        jax.ShapeDtypeStruct((1, _D), jnp.float32),
    ]
    in_specs = [
        pl.BlockSpec((_NC, _BN_ROWS, _D), lambda i: (0, i, 0)),
        pl.BlockSpec((_BN_ROWS, _D), lambda i: (i, 0)),
        pl.BlockSpec((1, _D), lambda i: (0, 0)),
    ]
    args = [parts, dinv, b]
    body = _combine_body
    if res is not None:
        in_specs.append(pl.BlockSpec((_BN_ROWS, _D), lambda i: (i, 0)))
        args.append(res)
        body = _combine_res_body
    return pl.pallas_call(
        body,
        grid=(_GRID,),
        in_specs=in_specs,
        out_specs=[
            pl.BlockSpec((_BN_ROWS, _D), lambda i: (i, 0)),
            pl.BlockSpec((1, _D), lambda i: (0, 0)),
            pl.BlockSpec((1, _D), lambda i: (0, 0)),
        ],
        out_shape=out_shape,
    )(*args)


def _apply_mm_body(y_ref, s_ref, ss_ref, g_ref, be_ref, w_ref, dv_ref, o_ref):
    mu = s_ref[...] / _N
    var = ss_ref[...] / _N - mu * mu
    inv = lax.rsqrt(var + _EPS)
    h = jnp.maximum((y_ref[...] - mu) * inv * g_ref[...] + be_ref[...], 0.0)
    hw = lax.dot_general(h, w_ref[...], (((1,), (1,)), ((), ())),
                         preferred_element_type=jnp.float32)
    o_ref[...] = hw * dv_ref[...]


def _apply_mm_call(y, s_, ss_, g, be, w, dinv):
    return pl.pallas_call(
        _apply_mm_body,
        grid=(_GRID,),
        in_specs=[
            pl.BlockSpec((_BN_ROWS, _D), lambda i: (i, 0)),
            pl.BlockSpec((1, _D), lambda i: (0, 0)),
            pl.BlockSpec((1, _D), lambda i: (0, 0)),
            pl.BlockSpec((1, _D), lambda i: (0, 0)),
            pl.BlockSpec((1, _D), lambda i: (0, 0)),
            pl.BlockSpec((_D, _D), lambda i: (0, 0)),
            pl.BlockSpec((_BN_ROWS, _D), lambda i: (i, 0)),
        ],
        out_specs=pl.BlockSpec((_BN_ROWS, _D), lambda i: (i, 0)),
        out_shape=jax.ShapeDtypeStruct((_N, _D), jnp.float32),
    )(y, s_, ss_, g, be, w, dinv)


def _apply_pred_body(y_ref, s_ref, ss_ref, g_ref, be_ref, w_ref, bp_ref, o_ref):
    mu = s_ref[...] / _N
    var = ss_ref[...] / _N - mu * mu
    inv = lax.rsqrt(var + _EPS)
    h = jnp.maximum((y_ref[...] - mu) * inv * g_ref[...] + be_ref[...], 0.0)
    hw = lax.dot_general(h, w_ref[...], (((1,), (1,)), ((), ())),
                         preferred_element_type=jnp.float32)
    o_ref[...] = hw + bp_ref[...]


def _apply_pred_call(y, s_, ss_, g, be, wp, bp):
    return pl.pallas_call(
        _apply_pred_body,
        grid=(_GRID,),
        in_specs=[
            pl.BlockSpec((_BN_ROWS, _D), lambda i: (i, 0)),
            pl.BlockSpec((1, _D), lambda i: (0, 0)),
            pl.BlockSpec((1, _D), lambda i: (0, 0)),
            pl.BlockSpec((1, _D), lambda i: (0, 0)),
            pl.BlockSpec((1, _D), lambda i: (0, 0)),
            pl.BlockSpec((_D, _D), lambda i: (0, 0)),
            pl.BlockSpec((1, _D), lambda i: (0, 0)),
        ],
        out_specs=pl.BlockSpec((_BN_ROWS, _D), lambda i: (i, 0)),
        out_shape=jax.ShapeDtypeStruct((_N, _D), jnp.float32),
    )(y, s_, ss_, g, be, wp, bp)


def kernel(x, edge_index, W_enc, b_enc, Ws, bs, gammas, betas, W_pred, b_pred):
    colw = edge_index[1].reshape(_NW, _NCHUNK, _K)
    eiw = (
        edge_index.reshape(2, _NW, _NCHUNK, _K)
        .transpose(1, 2, 0, 3)
        .reshape(_NW, 2 * _NCHUNK, _K)
    )

    dummy = jnp.zeros((_K, _D), jnp.float32)
    deg_parts = _sc_degree(colw, dummy)
    dinv = _dinv_call(deg_parts)

    h = _enc_call(x, W_enc, b_enc.reshape(1, _D))
    hp = _smm_call(h, Ws[0], dinv)

    wp_pad = jnp.zeros((_D, _D), jnp.float32).at[:_C].set(W_pred)
    bp_pad = jnp.zeros((1, _D), jnp.float32).at[0, :_C].set(b_pred)

    y_last = None
    out = None
    for i in range(_L):
        parts = _sc_aggregate(hp, eiw, dummy)
        y, s_, ss_ = _combine_call(parts, dinv, bs[i].reshape(1, _D), y_last)
        y_last = y
        g = gammas[i].reshape(1, _D)
        be = betas[i].reshape(1, _D)
        if i < _L - 1:
            hp = _apply_mm_call(y, s_, ss_, g, be, Ws[i + 1], dinv)
        else:
            out = _apply_pred_call(y, s_, ss_, g, be, wp_pad, bp_pad)
    return out[:, :_C]


# speedup vs baseline: 16.3396x; 1.0290x over previous
import functools

import jax
import jax.numpy as jnp
from jax import lax
from jax.experimental import pallas as pl
from jax.experimental.pallas import tpu as pltpu
from jax.experimental.pallas import tpu_sc as plsc

_N = 10000
_E = 320000
_D = 128
_C = 40
_L = 3
_EPS = 1e-5

_NC = 2
_NS = 16
_NW = _NC * _NS
_EPW = _E // _NW
_K = 125
_NCHUNK = _EPW // _K
_NP = 10240
_RPT = _NP // _NS
_ZROWS = 128

_mesh = plsc.VectorSubcoreMesh(core_axis_name="c", subcore_axis_name="s")

_ZC = 80


@functools.partial(
    pl.kernel,
    out_type=jax.ShapeDtypeStruct((_NC, _NP, _D), jnp.float32),
    mesh=_mesh,
    scratch_types=[
        pltpu.VMEM((_NCHUNK, _K), jnp.int32),
        pltpu.VMEM((_K, _D), jnp.float32),
        pltpu.VMEM_SHARED((_NP, _D), jnp.float32),
        pltpu.SemaphoreType.DMA,
    ],
)
def _sc_degree(colw_hbm, dummy_hbm, out_hbm, idx_v, ones_v, acc_sh, sem):
    c = lax.axis_index("c")
    s = lax.axis_index("s")
    w = c * _NS + s

    pltpu.async_copy(colw_hbm.at[w], idx_v, sem)

    def _fill(val):
        def _f(i, carry):
            for j in range(_D // 16):
                ones_v[i, pl.ds(j * 16, 16)] = jnp.full((16,), val, jnp.float32)
            return carry
        return _f

    lax.fori_loop(0, _ZC, _fill(0.0), 0)

    def _zero_acc(k, carry):
        pltpu.sync_copy(ones_v.at[pl.ds(0, _ZC)],
                        acc_sh.at[pl.ds(s * _RPT + k * _ZC, _ZC)])
        return carry

    lax.fori_loop(0, _RPT // _ZC, _zero_acc, 0)
    lax.fori_loop(0, _K, _fill(1.0), 0)
    pltpu.make_async_copy(colw_hbm.at[w], idx_v, sem).wait()
    plsc.subcore_barrier()

    _W = 8

    def _step(i, carry):
        pltpu.async_copy(ones_v, acc_sh.at[idx_v.at[i]], sem, add=True)

        @pl.when(i >= _W)
        def _():
            pltpu.make_async_copy(dummy_hbm, ones_v, sem).wait()

        return carry

    lax.fori_loop(0, _NCHUNK, _step, 0)

    def _drain(i, carry):
        pltpu.make_async_copy(dummy_hbm, ones_v, sem).wait()
        return carry

    lax.fori_loop(0, _W, _drain, 0)
    plsc.subcore_barrier()
    pltpu.sync_copy(
        acc_sh.at[pl.ds(s * _RPT, _RPT)], out_hbm.at[c, pl.ds(s * _RPT, _RPT)]
    )


_SCH = 8
_NSEG = _NCHUNK // _SCH


@functools.partial(
    pl.kernel,
    out_type=jax.ShapeDtypeStruct((_NC, _NP, _D), jnp.float32),
    mesh=_mesh,
    scratch_types=[
        pltpu.VMEM((2, _SCH, _K), jnp.int32),
        pltpu.VMEM((2, _SCH, _K), jnp.int32),
        pltpu.VMEM((_K, _D), jnp.float32),
        pltpu.VMEM((_K, _D), jnp.float32),
        pltpu.VMEM_SHARED((_NP, _D), jnp.float32),
        pltpu.SemaphoreType.DMA,
        pltpu.SemaphoreType.DMA,
        pltpu.SemaphoreType.DMA,
        pltpu.SemaphoreType.DMA,
        pltpu.SemaphoreType.DMA,
    ],
)
def _sc_aggregate(hp_hbm, roww_hbm, colw_hbm, dummy_hbm, out_hbm, ri_v, ci_v,
                  buf0, buf1, acc_sh, sg0, sg1, ss0, ss1, si):
    c = lax.axis_index("c")
    s = lax.axis_index("s")
    w = c * _NS + s

    pltpu.async_copy(roww_hbm.at[w, pl.ds(0, _SCH)], ri_v.at[0], si)
    pltpu.async_copy(colw_hbm.at[w, pl.ds(0, _SCH)], ci_v.at[0], si)

    def _fz(i, carry):
        for j in range(_D // 16):
            buf0[i, pl.ds(j * 16, 16)] = jnp.zeros((16,), jnp.float32)
        return carry

    lax.fori_loop(0, _ZC, _fz, 0)

    def _zero_acc(k, carry):
        pltpu.sync_copy(buf0.at[pl.ds(0, _ZC)],
                        acc_sh.at[pl.ds(s * _RPT + k * _ZC, _ZC)])
        return carry

    lax.fori_loop(0, _RPT // _ZC, _zero_acc, 0)
    pltpu.make_async_copy(roww_hbm.at[w, pl.ds(0, _SCH)], ri_v.at[0], si).wait()
    pltpu.make_async_copy(colw_hbm.at[w, pl.ds(0, _SCH)], ci_v.at[0], si).wait()
    plsc.subcore_barrier()

    def _wait_gather(buf, sg):
        pltpu.make_async_copy(dummy_hbm, buf, sg).wait()

    def _drain_scatter(buf, ss):
        pltpu.make_async_copy(dummy_hbm, buf, ss).wait()

    def _seg(h, carry):
        par = lax.rem(h, 2)
        pnext = 1 - par

        @pl.when(h + 1 < _NSEG)
        def _():
            pltpu.async_copy(
                roww_hbm.at[w, pl.ds((h + 1) * _SCH, _SCH)], ri_v.at[pnext], si)
            pltpu.async_copy(
                colw_hbm.at[w, pl.ds((h + 1) * _SCH, _SCH)], ci_v.at[pnext], si)

        for j0 in range(0, _SCH, 2):
            j1 = j0 + 1
            if j0 >= 2:
                _drain_scatter(buf0, ss0)
            else:
                @pl.when(h > 0)
                def _():
                    _drain_scatter(buf0, ss0)
            pltpu.async_copy(hp_hbm.at[ri_v.at[par, j0]], buf0, sg0)
            if j0 >= 2:
                _drain_scatter(buf1, ss1)
            else:
                @pl.when(h > 0)
                def _():
                    _drain_scatter(buf1, ss1)
            pltpu.async_copy(hp_hbm.at[ri_v.at[par, j1]], buf1, sg1)
            _wait_gather(buf0, sg0)
            pltpu.async_copy(buf0, acc_sh.at[ci_v.at[par, j0]], ss0, add=True)
            _wait_gather(buf1, sg1)
            pltpu.async_copy(buf1, acc_sh.at[ci_v.at[par, j1]], ss1, add=True)

        @pl.when(h + 1 < _NSEG)
        def _():
            pltpu.make_async_copy(
                roww_hbm.at[w, pl.ds(0, _SCH)], ri_v.at[pnext], si).wait()
            pltpu.make_async_copy(
                colw_hbm.at[w, pl.ds(0, _SCH)], ci_v.at[pnext], si).wait()

        return carry

    lax.fori_loop(0, _NSEG, _seg, 0)
    _drain_scatter(buf0, ss0)
    _drain_scatter(buf1, ss1)
    plsc.subcore_barrier()
    pltpu.sync_copy(
        acc_sh.at[pl.ds(s * _RPT, _RPT)], out_hbm.at[c, pl.ds(s * _RPT, _RPT)]
    )


_BN_ROWS = 1000
_GRID = _N // _BN_ROWS


def _pre_body(p_ref, x_ref, we_ref, be_ref, w0_ref, dv_ref, hp_ref):
    d = p_ref[0] + p_ref[1]
    dv = jnp.where(d > 0, lax.rsqrt(d), 0.0)
    dv_ref[...] = dv
    h = lax.dot_general(x_ref[...], we_ref[...], (((1,), (1,)), ((), ())),
                        preferred_element_type=jnp.float32)
    h = jnp.maximum(h + be_ref[...], 0.0)
    hw = lax.dot_general(h, w0_ref[...], (((1,), (1,)), ((), ())),
                         preferred_element_type=jnp.float32)
    hp_ref[...] = hw * dv


def _pre_call(parts, x, we, be, w0):
    return pl.pallas_call(
        _pre_body,
        grid=(_GRID,),
        in_specs=[
            pl.BlockSpec((_NC, _BN_ROWS, _D), lambda i: (0, i, 0)),
            pl.BlockSpec((_BN_ROWS, _D), lambda i: (i, 0)),
            pl.BlockSpec((_D, _D), lambda i: (0, 0)),
            pl.BlockSpec((1, _D), lambda i: (0, 0)),
            pl.BlockSpec((_D, _D), lambda i: (0, 0)),
        ],
        out_specs=[
            pl.BlockSpec((_BN_ROWS, _D), lambda i: (i, 0)),
            pl.BlockSpec((_BN_ROWS, _D), lambda i: (i, 0)),
        ],
        out_shape=[
            jax.ShapeDtypeStruct((_N, _D), jnp.float32),
            jax.ShapeDtypeStruct((_N, _D), jnp.float32),
        ],
    )(parts, x, we, be, w0)


def _combine_body(p_ref, dv_ref, b_ref, y_ref, s_ref, ss_ref):
    i = pl.program_id(0)
    y = (p_ref[0] + p_ref[1]) * dv_ref[...] + b_ref[...]
    y_ref[...] = y

    @pl.when(i == 0)
    def _():
        s_ref[...] = jnp.zeros_like(s_ref)
        ss_ref[...] = jnp.zeros_like(ss_ref)

    s_ref[...] += jnp.sum(y, axis=0, keepdims=True)
    ss_ref[...] += jnp.sum(y * y, axis=0, keepdims=True)


def _combine_res_body(p_ref, dv_ref, b_ref, r_ref, y_ref, s_ref, ss_ref):
    i = pl.program_id(0)
    y = (p_ref[0] + p_ref[1]) * dv_ref[...] + b_ref[...] + r_ref[...]
    y_ref[...] = y

    @pl.when(i == 0)
    def _():
        s_ref[...] = jnp.zeros_like(s_ref)
        ss_ref[...] = jnp.zeros_like(ss_ref)

    s_ref[...] += jnp.sum(y, axis=0, keepdims=True)
    ss_ref[...] += jnp.sum(y * y, axis=0, keepdims=True)


def _combine_call(parts, dinv, b, res=None):
    out_shape = [
        jax.ShapeDtypeStruct((_N, _D), jnp.float32),
        jax.ShapeDtypeStruct((1, _D), jnp.float32),
        jax.ShapeDtypeStruct((1, _D), jnp.float32),
    ]
    in_specs = [
        pl.BlockSpec((_NC, _BN_ROWS, _D), lambda i: (0, i, 0)),
        pl.BlockSpec((_BN_ROWS, _D), lambda i: (i, 0)),
        pl.BlockSpec((1, _D), lambda i: (0, 0)),
    ]
    args = [parts, dinv, b]
    body = _combine_body
    if res is not None:
        in_specs.append(pl.BlockSpec((_BN_ROWS, _D), lambda i: (i, 0)))
        args.append(res)
        body = _combine_res_body
    return pl.pallas_call(
        body,
        grid=(_GRID,),
        in_specs=in_specs,
        out_specs=[
            pl.BlockSpec((_BN_ROWS, _D), lambda i: (i, 0)),
            pl.BlockSpec((1, _D), lambda i: (0, 0)),
            pl.BlockSpec((1, _D), lambda i: (0, 0)),
        ],
        out_shape=out_shape,
    )(*args)


def _apply_mm_body(y_ref, s_ref, ss_ref, g_ref, be_ref, w_ref, dv_ref, o_ref):
    mu = s_ref[...] / _N
    var = ss_ref[...] / _N - mu * mu
    inv = lax.rsqrt(var + _EPS)
    h = jnp.maximum((y_ref[...] - mu) * inv * g_ref[...] + be_ref[...], 0.0)
    hw = lax.dot_general(h, w_ref[...], (((1,), (1,)), ((), ())),
                         preferred_element_type=jnp.float32)
    o_ref[...] = hw * dv_ref[...]


def _apply_mm_call(y, s_, ss_, g, be, w, dinv):
    return pl.pallas_call(
        _apply_mm_body,
        grid=(_GRID,),
        in_specs=[
            pl.BlockSpec((_BN_ROWS, _D), lambda i: (i, 0)),
            pl.BlockSpec((1, _D), lambda i: (0, 0)),
            pl.BlockSpec((1, _D), lambda i: (0, 0)),
            pl.BlockSpec((1, _D), lambda i: (0, 0)),
            pl.BlockSpec((1, _D), lambda i: (0, 0)),
            pl.BlockSpec((_D, _D), lambda i: (0, 0)),
            pl.BlockSpec((_BN_ROWS, _D), lambda i: (i, 0)),
        ],
        out_specs=pl.BlockSpec((_BN_ROWS, _D), lambda i: (i, 0)),
        out_shape=jax.ShapeDtypeStruct((_N, _D), jnp.float32),
    )(y, s_, ss_, g, be, w, dinv)


def _apply_pred_body(y_ref, s_ref, ss_ref, g_ref, be_ref, w_ref, bp_ref, o_ref):
    mu = s_ref[...] / _N
    var = ss_ref[...] / _N - mu * mu
    inv = lax.rsqrt(var + _EPS)
    h = jnp.maximum((y_ref[...] - mu) * inv * g_ref[...] + be_ref[...], 0.0)
    hw = lax.dot_general(h, w_ref[...], (((1,), (1,)), ((), ())),
                         preferred_element_type=jnp.float32)
    o_ref[...] = hw + bp_ref[...]


def _apply_pred_call(y, s_, ss_, g, be, wp, bp):
    return pl.pallas_call(
        _apply_pred_body,
        grid=(_GRID,),
        in_specs=[
            pl.BlockSpec((_BN_ROWS, _D), lambda i: (i, 0)),
            pl.BlockSpec((1, _D), lambda i: (0, 0)),
            pl.BlockSpec((1, _D), lambda i: (0, 0)),
            pl.BlockSpec((1, _D), lambda i: (0, 0)),
            pl.BlockSpec((1, _D), lambda i: (0, 0)),
            pl.BlockSpec((_D, _D), lambda i: (0, 0)),
            pl.BlockSpec((1, _D), lambda i: (0, 0)),
        ],
        out_specs=pl.BlockSpec((_BN_ROWS, _D), lambda i: (i, 0)),
        out_shape=jax.ShapeDtypeStruct((_N, _D), jnp.float32),
    )(y, s_, ss_, g, be, wp, bp)


def kernel(x, edge_index, W_enc, b_enc, Ws, bs, gammas, betas, W_pred, b_pred):
    roww = edge_index[0].reshape(_NW, _NCHUNK, _K)
    colw = edge_index[1].reshape(_NW, _NCHUNK, _K)

    dummy = jnp.zeros((_K, _D), jnp.float32)
    deg_parts = _sc_degree(colw, dummy)
    dinv, hp = _pre_call(deg_parts, x, W_enc, b_enc.reshape(1, _D), Ws[0])

    wp_pad = jnp.zeros((_D, _D), jnp.float32).at[:_C].set(W_pred)
    bp_pad = jnp.zeros((1, _D), jnp.float32).at[0, :_C].set(b_pred)

    y_last = None
    out = None
    for i in range(_L):
        parts = _sc_aggregate(hp, roww, colw, dummy)
        y, s_, ss_ = _combine_call(parts, dinv, bs[i].reshape(1, _D), y_last)
        y_last = y
        g = gammas[i].reshape(1, _D)
        be = betas[i].reshape(1, _D)
        if i < _L - 1:
            hp = _apply_mm_call(y, s_, ss_, g, be, Ws[i + 1], dinv)
        else:
            out = _apply_pred_call(y, s_, ss_, g, be, wp_pad, bp_pad)
    return out[:, :_C]


# speedup vs baseline: 17.2035x; 1.0529x over previous
import functools

import jax
import jax.numpy as jnp
from jax import lax
from jax.experimental import pallas as pl
from jax.experimental.pallas import tpu as pltpu
from jax.experimental.pallas import tpu_sc as plsc

_N = 10000
_E = 320000
_D = 128
_C = 40
_L = 3
_EPS = 1e-5

_NC = 2
_NS = 16
_NW = _NC * _NS
_EPW = _E // _NW
_K = 125
_NCHUNK = _EPW // _K
_NP = 10240
_RPT = _NP // _NS
_ZROWS = 128

_mesh = plsc.VectorSubcoreMesh(core_axis_name="c", subcore_axis_name="s")

_ZC = 80


@functools.partial(
    pl.kernel,
    out_type=jax.ShapeDtypeStruct((_NC, _NP, _D), jnp.float32),
    mesh=_mesh,
    scratch_types=[
        pltpu.VMEM((_NCHUNK, _K), jnp.int32),
        pltpu.VMEM((_K, _D), jnp.float32),
        pltpu.VMEM_SHARED((_NP, _D), jnp.float32),
        pltpu.SemaphoreType.DMA,
    ],
)
def _sc_degree(colw_hbm, dummy_hbm, out_hbm, idx_v, ones_v, acc_sh, sem):
    c = lax.axis_index("c")
    s = lax.axis_index("s")
    w = c * _NS + s

    pltpu.async_copy(colw_hbm.at[w], idx_v, sem)

    def _fill(val):
        def _f(i, carry):
            for j in range(_D // 16):
                ones_v[i, pl.ds(j * 16, 16)] = jnp.full((16,), val, jnp.float32)
            return carry
        return _f

    lax.fori_loop(0, _ZC, _fill(0.0), 0)

    def _zero_acc(k, carry):
        pltpu.sync_copy(ones_v.at[pl.ds(0, _ZC)],
                        acc_sh.at[pl.ds(s * _RPT + k * _ZC, _ZC)])
        return carry

    lax.fori_loop(0, _RPT // _ZC, _zero_acc, 0)
    lax.fori_loop(0, _K, _fill(1.0), 0)
    pltpu.make_async_copy(colw_hbm.at[w], idx_v, sem).wait()
    plsc.subcore_barrier()

    _W = 8

    def _step(i, carry):
        pltpu.async_copy(ones_v, acc_sh.at[idx_v.at[i]], sem, add=True)

        @pl.when(i >= _W)
        def _():
            pltpu.make_async_copy(dummy_hbm, ones_v, sem).wait()

        return carry

    lax.fori_loop(0, _NCHUNK, _step, 0)

    def _drain(i, carry):
        pltpu.make_async_copy(dummy_hbm, ones_v, sem).wait()
        return carry

    lax.fori_loop(0, _W, _drain, 0)
    plsc.subcore_barrier()
    pltpu.sync_copy(
        acc_sh.at[pl.ds(s * _RPT, _RPT)], out_hbm.at[c, pl.ds(s * _RPT, _RPT)]
    )


_SCH = 8
_NSEG = _NCHUNK // _SCH


@functools.partial(
    pl.kernel,
    out_type=jax.ShapeDtypeStruct((_NC, _NP, _D), jnp.float32),
    mesh=_mesh,
    scratch_types=[
        pltpu.VMEM((2, _SCH, _K), jnp.int32),
        pltpu.VMEM((2, _SCH, _K), jnp.int32),
        pltpu.VMEM((_K, _D), jnp.float32),
        pltpu.VMEM((_K, _D), jnp.float32),
        pltpu.VMEM_SHARED((_NP, _D), jnp.float32),
        pltpu.SemaphoreType.DMA,
        pltpu.SemaphoreType.DMA,
        pltpu.SemaphoreType.DMA,
        pltpu.SemaphoreType.DMA,
        pltpu.SemaphoreType.DMA,
    ],
)
def _sc_aggregate(hp_hbm, roww_hbm, colw_hbm, dummy_hbm, out_hbm, ri_v, ci_v,
                  buf0, buf1, acc_sh, sg0, sg1, ss0, ss1, si):
    c = lax.axis_index("c")
    s = lax.axis_index("s")
    w = c * _NS + s

    pltpu.async_copy(roww_hbm.at[w, pl.ds(0, _SCH)], ri_v.at[0], si)
    pltpu.async_copy(colw_hbm.at[w, pl.ds(0, _SCH)], ci_v.at[0], si)

    def _fz(i, carry):
        for j in range(_D // 16):
            buf0[i, pl.ds(j * 16, 16)] = jnp.zeros((16,), jnp.float32)
        return carry

    lax.fori_loop(0, _ZC, _fz, 0)

    def _zero_acc(k, carry):
        pltpu.sync_copy(buf0.at[pl.ds(0, _ZC)],
                        acc_sh.at[pl.ds(s * _RPT + k * _ZC, _ZC)])
        return carry

    lax.fori_loop(0, _RPT // _ZC, _zero_acc, 0)
    pltpu.make_async_copy(roww_hbm.at[w, pl.ds(0, _SCH)], ri_v.at[0], si).wait()
    pltpu.make_async_copy(colw_hbm.at[w, pl.ds(0, _SCH)], ci_v.at[0], si).wait()
    plsc.subcore_barrier()

    def _wait_gather(buf, sg):
        pltpu.make_async_copy(dummy_hbm, buf, sg).wait()

    def _drain_scatter(buf, ss):
        pltpu.make_async_copy(dummy_hbm, buf, ss).wait()

    def _seg(h, carry):
        par = lax.rem(h, 2)
        pnext = 1 - par

        @pl.when(h + 1 < _NSEG)
        def _():
            pltpu.async_copy(
                roww_hbm.at[w, pl.ds((h + 1) * _SCH, _SCH)], ri_v.at[pnext], si)
            pltpu.async_copy(
                colw_hbm.at[w, pl.ds((h + 1) * _SCH, _SCH)], ci_v.at[pnext], si)

        for j0 in range(0, _SCH, 2):
            j1 = j0 + 1
            if j0 >= 2:
                _drain_scatter(buf0, ss0)
            else:
                @pl.when(h > 0)
                def _():
                    _drain_scatter(buf0, ss0)
            pltpu.async_copy(hp_hbm.at[ri_v.at[par, j0]], buf0, sg0)
            if j0 >= 2:
                _drain_scatter(buf1, ss1)
            else:
                @pl.when(h > 0)
                def _():
                    _drain_scatter(buf1, ss1)
            pltpu.async_copy(hp_hbm.at[ri_v.at[par, j1]], buf1, sg1)
            _wait_gather(buf0, sg0)
            pltpu.async_copy(buf0, acc_sh.at[ci_v.at[par, j0]], ss0, add=True)
            _wait_gather(buf1, sg1)
            pltpu.async_copy(buf1, acc_sh.at[ci_v.at[par, j1]], ss1, add=True)

        @pl.when(h + 1 < _NSEG)
        def _():
            pltpu.make_async_copy(
                roww_hbm.at[w, pl.ds(0, _SCH)], ri_v.at[pnext], si).wait()
            pltpu.make_async_copy(
                colw_hbm.at[w, pl.ds(0, _SCH)], ci_v.at[pnext], si).wait()

        return carry

    lax.fori_loop(0, _NSEG, _seg, 0)
    _drain_scatter(buf0, ss0)
    _drain_scatter(buf1, ss1)
    plsc.subcore_barrier()
    pltpu.sync_copy(
        acc_sh.at[pl.ds(s * _RPT, _RPT)], out_hbm.at[c, pl.ds(s * _RPT, _RPT)]
    )


_BN_ROWS = 1000
_GRID = _N // _BN_ROWS


def _pre_body(p_ref, x_ref, we_ref, be_ref, w0_ref, dv_ref, hp_ref):
    d = p_ref[0] + p_ref[1]
    dv = jnp.where(d > 0, lax.rsqrt(d), 0.0)
    dv_ref[...] = dv
    h = lax.dot_general(x_ref[...], we_ref[...], (((1,), (1,)), ((), ())),
                        preferred_element_type=jnp.float32)
    h = jnp.maximum(h + be_ref[...], 0.0)
    hw = lax.dot_general(h, w0_ref[...], (((1,), (1,)), ((), ())),
                         preferred_element_type=jnp.float32)
    hp_ref[...] = hw * dv


def _pre_call(parts, x, we, be, w0):
    return pl.pallas_call(
        _pre_body,
        grid=(_GRID,),
        in_specs=[
            pl.BlockSpec((_NC, _BN_ROWS, _D), lambda i: (0, i, 0)),
            pl.BlockSpec((_BN_ROWS, _D), lambda i: (i, 0)),
            pl.BlockSpec((_D, _D), lambda i: (0, 0)),
            pl.BlockSpec((1, _D), lambda i: (0, 0)),
            pl.BlockSpec((_D, _D), lambda i: (0, 0)),
        ],
        out_specs=[
            pl.BlockSpec((_BN_ROWS, _D), lambda i: (i, 0)),
            pl.BlockSpec((_BN_ROWS, _D), lambda i: (i, 0)),
        ],
        out_shape=[
            jax.ShapeDtypeStruct((_N, _D), jnp.float32),
            jax.ShapeDtypeStruct((_N, _D), jnp.float32),
        ],
    )(parts, x, we, be, w0)


def _layer_mm_body(p_ref, dv_ref, b_ref, r_ref, g_ref, be_ref, w_ref,
                   y_ref, hp_ref):
    y = (p_ref[0, :_N] + p_ref[1, :_N]) * dv_ref[...] + b_ref[...] + r_ref[...]
    y_ref[...] = y
    mu = jnp.mean(y, axis=0, keepdims=True)
    var = jnp.mean(y * y, axis=0, keepdims=True) - mu * mu
    h = jnp.maximum(
        (y - mu) * lax.rsqrt(var + _EPS) * g_ref[...] + be_ref[...], 0.0)
    hw = lax.dot_general(h, w_ref[...], (((1,), (1,)), ((), ())),
                         preferred_element_type=jnp.float32)
    hp_ref[...] = hw * dv_ref[...]


def _layer_mm0_body(p_ref, dv_ref, b_ref, g_ref, be_ref, w_ref, y_ref, hp_ref):
    y = (p_ref[0, :_N] + p_ref[1, :_N]) * dv_ref[...] + b_ref[...]
    y_ref[...] = y
    mu = jnp.mean(y, axis=0, keepdims=True)
    var = jnp.mean(y * y, axis=0, keepdims=True) - mu * mu
    h = jnp.maximum(
        (y - mu) * lax.rsqrt(var + _EPS) * g_ref[...] + be_ref[...], 0.0)
    hw = lax.dot_general(h, w_ref[...], (((1,), (1,)), ((), ())),
                         preferred_element_type=jnp.float32)
    hp_ref[...] = hw * dv_ref[...]


def _layer_call(parts, dinv, b, res, g, be, w_next):
    out_shape = [
        jax.ShapeDtypeStruct((_N, _D), jnp.float32),
        jax.ShapeDtypeStruct((_N, _D), jnp.float32),
    ]
    if res is None:
        return pl.pallas_call(_layer_mm0_body, out_shape=out_shape)(
            parts, dinv, b, g, be, w_next)
    return pl.pallas_call(_layer_mm_body, out_shape=out_shape)(
        parts, dinv, b, res, g, be, w_next)


def _layer_pred_body(p_ref, dv_ref, b_ref, r_ref, g_ref, be_ref, w_ref,
                     bp_ref, o_ref):
    y = (p_ref[0, :_N] + p_ref[1, :_N]) * dv_ref[...] + b_ref[...] + r_ref[...]
    mu = jnp.mean(y, axis=0, keepdims=True)
    var = jnp.mean(y * y, axis=0, keepdims=True) - mu * mu
    h = jnp.maximum(
        (y - mu) * lax.rsqrt(var + _EPS) * g_ref[...] + be_ref[...], 0.0)
    hw = lax.dot_general(h, w_ref[...], (((1,), (1,)), ((), ())),
                         preferred_element_type=jnp.float32)
    o_ref[...] = hw + bp_ref[...]


def _layer_pred_call(parts, dinv, b, res, g, be, wp, bp):
    return pl.pallas_call(
        _layer_pred_body,
        out_shape=jax.ShapeDtypeStruct((_N, _D), jnp.float32),
    )(parts, dinv, b, res, g, be, wp, bp)


def kernel(x, edge_index, W_enc, b_enc, Ws, bs, gammas, betas, W_pred, b_pred):
    roww = edge_index[0].reshape(_NW, _NCHUNK, _K)
    colw = edge_index[1].reshape(_NW, _NCHUNK, _K)

    dummy = jnp.zeros((_K, _D), jnp.float32)
    deg_parts = _sc_degree(colw, dummy)
    dinv, hp = _pre_call(deg_parts, x, W_enc, b_enc.reshape(1, _D), Ws[0])

    wp_pad = jnp.zeros((_D, _D), jnp.float32).at[:_C].set(W_pred)
    bp_pad = jnp.zeros((1, _D), jnp.float32).at[0, :_C].set(b_pred)

    y_last = None
    out = None
    for i in range(_L):
        parts = _sc_aggregate(hp, roww, colw, dummy)
        b = bs[i].reshape(1, _D)
        g = gammas[i].reshape(1, _D)
        be = betas[i].reshape(1, _D)
        if i < _L - 1:
            y, hp = _layer_call(parts, dinv, b, y_last, g, be, Ws[i + 1])
            y_last = y
        else:
            out = _layer_pred_call(parts, dinv, b, y_last, g, be, wp_pad,
                                   bp_pad)
    return out[:, :_C]


# speedup vs baseline: 17.2828x; 1.0046x over previous
import functools

import jax
import jax.numpy as jnp
from jax import lax
from jax.experimental import pallas as pl
from jax.experimental.pallas import tpu as pltpu
from jax.experimental.pallas import tpu_sc as plsc

_N = 10000
_E = 320000
_D = 128
_C = 40
_L = 3
_EPS = 1e-5

_NC = 2
_NS = 16
_NW = _NC * _NS
_EPW = _E // _NW
_K = 125
_NCHUNK = _EPW // _K
_NP = 10240
_RPT = _NP // _NS
_ZROWS = 128

_mesh = plsc.VectorSubcoreMesh(core_axis_name="c", subcore_axis_name="s")

_ZC = 80


@functools.partial(
    pl.kernel,
    out_type=jax.ShapeDtypeStruct((_NC, _NP, _D), jnp.float32),
    mesh=_mesh,
    scratch_types=[
        pltpu.VMEM((_NCHUNK, _K), jnp.int32),
        pltpu.VMEM((_K, _D), jnp.float32),
        pltpu.VMEM_SHARED((_NP, _D), jnp.float32),
        pltpu.SemaphoreType.DMA,
    ],
)
def _sc_degree(colw_hbm, dummy_hbm, out_hbm, idx_v, ones_v, acc_sh, sem):
    c = lax.axis_index("c")
    s = lax.axis_index("s")
    w = c * _NS + s

    pltpu.async_copy(colw_hbm.at[w], idx_v, sem)

    def _fill(val):
        def _f(i, carry):
            for j in range(_D // 16):
                ones_v[i, pl.ds(j * 16, 16)] = jnp.full((16,), val, jnp.float32)
            return carry
        return _f

    lax.fori_loop(0, _ZC, _fill(0.0), 0)

    def _zero_acc(k, carry):
        pltpu.sync_copy(ones_v.at[pl.ds(0, _ZC)],
                        acc_sh.at[pl.ds(s * _RPT + k * _ZC, _ZC)])
        return carry

    lax.fori_loop(0, _RPT // _ZC, _zero_acc, 0)
    lax.fori_loop(0, _K, _fill(1.0), 0)
    pltpu.make_async_copy(colw_hbm.at[w], idx_v, sem).wait()
    plsc.subcore_barrier()

    _W = 8

    def _step(i, carry):
        pltpu.async_copy(ones_v, acc_sh.at[idx_v.at[i]], sem, add=True)

        @pl.when(i >= _W)
        def _():
            pltpu.make_async_copy(dummy_hbm, ones_v, sem).wait()

        return carry

    lax.fori_loop(0, _NCHUNK, _step, 0)

    def _drain(i, carry):
        pltpu.make_async_copy(dummy_hbm, ones_v, sem).wait()
        return carry

    lax.fori_loop(0, _W, _drain, 0)
    plsc.subcore_barrier()
    pltpu.sync_copy(
        acc_sh.at[pl.ds(s * _RPT, _RPT)], out_hbm.at[c, pl.ds(s * _RPT, _RPT)]
    )


_SCH = 8
_NSEG = _NCHUNK // _SCH


@functools.partial(
    pl.kernel,
    out_type=jax.ShapeDtypeStruct((_NC, _NP, _D), jnp.float32),
    mesh=_mesh,
    scratch_types=[
        pltpu.VMEM((2, _SCH, _K), jnp.int32),
        pltpu.VMEM((2, _SCH, _K), jnp.int32),
        pltpu.VMEM((_K, _D), jnp.float32),
        pltpu.VMEM((_K, _D), jnp.float32),
        pltpu.VMEM_SHARED((_NP, _D), jnp.float32),
        pltpu.SemaphoreType.DMA,
        pltpu.SemaphoreType.DMA,
        pltpu.SemaphoreType.DMA,
        pltpu.SemaphoreType.DMA,
        pltpu.SemaphoreType.DMA,
    ],
)
def _sc_aggregate(hp_hbm, roww_hbm, colw_hbm, dummy_hbm, out_hbm, ri_v, ci_v,
                  buf0, buf1, acc_sh, sg0, sg1, ss0, ss1, si):
    c = lax.axis_index("c")
    s = lax.axis_index("s")
    w = c * _NS + s

    pltpu.async_copy(roww_hbm.at[w, pl.ds(0, _SCH)], ri_v.at[0], si)
    pltpu.async_copy(colw_hbm.at[w, pl.ds(0, _SCH)], ci_v.at[0], si)

    def _fz(i, carry):
        for j in range(_D // 16):
            buf0[i, pl.ds(j * 16, 16)] = jnp.zeros((16,), jnp.float32)
        return carry

    lax.fori_loop(0, _ZC, _fz, 0)

    def _zero_acc(k, carry):
        pltpu.sync_copy(buf0.at[pl.ds(0, _ZC)],
                        acc_sh.at[pl.ds(s * _RPT + k * _ZC, _ZC)])
        return carry

    lax.fori_loop(0, _RPT // _ZC, _zero_acc, 0)
    pltpu.make_async_copy(roww_hbm.at[w, pl.ds(0, _SCH)], ri_v.at[0], si).wait()
    pltpu.make_async_copy(colw_hbm.at[w, pl.ds(0, _SCH)], ci_v.at[0], si).wait()
    plsc.subcore_barrier()

    def _wait_gather(buf, sg):
        pltpu.make_async_copy(dummy_hbm, buf, sg).wait()

    def _drain_scatter(buf, ss):
        pltpu.make_async_copy(dummy_hbm, buf, ss).wait()

    def _seg(h, carry):
        par = lax.rem(h, 2)
        pnext = 1 - par

        @pl.when(h + 1 < _NSEG)
        def _():
            pltpu.async_copy(
                roww_hbm.at[w, pl.ds((h + 1) * _SCH, _SCH)], ri_v.at[pnext], si)
            pltpu.async_copy(
                colw_hbm.at[w, pl.ds((h + 1) * _SCH, _SCH)], ci_v.at[pnext], si)

        for j0 in range(0, _SCH, 2):
            j1 = j0 + 1
            if j0 >= 2:
                _drain_scatter(buf0, ss0)
            else:
                @pl.when(h > 0)
                def _():
                    _drain_scatter(buf0, ss0)
            pltpu.async_copy(hp_hbm.at[ri_v.at[par, j0]], buf0, sg0)
            if j0 >= 2:
                _drain_scatter(buf1, ss1)
            else:
                @pl.when(h > 0)
                def _():
                    _drain_scatter(buf1, ss1)
            pltpu.async_copy(hp_hbm.at[ri_v.at[par, j1]], buf1, sg1)
            _wait_gather(buf0, sg0)
            pltpu.async_copy(buf0, acc_sh.at[ci_v.at[par, j0]], ss0, add=True)
            _wait_gather(buf1, sg1)
            pltpu.async_copy(buf1, acc_sh.at[ci_v.at[par, j1]], ss1, add=True)

        @pl.when(h + 1 < _NSEG)
        def _():
            pltpu.make_async_copy(
                roww_hbm.at[w, pl.ds(0, _SCH)], ri_v.at[pnext], si).wait()
            pltpu.make_async_copy(
                colw_hbm.at[w, pl.ds(0, _SCH)], ci_v.at[pnext], si).wait()

        return carry

    lax.fori_loop(0, _NSEG, _seg, 0)
    _drain_scatter(buf0, ss0)
    _drain_scatter(buf1, ss1)
    plsc.subcore_barrier()
    pltpu.sync_copy(
        acc_sh.at[pl.ds(s * _RPT, _RPT)], out_hbm.at[c, pl.ds(s * _RPT, _RPT)]
    )


_BN_ROWS = 1000
_GRID = _N // _BN_ROWS


def _pre_body(p_ref, x_ref, we_ref, be_ref, w0_ref, dv_ref, hp_ref):
    d = p_ref[0, :_N] + p_ref[1, :_N]
    dv = jnp.where(d > 0, lax.rsqrt(d), 0.0)
    dv_ref[...] = dv
    h = lax.dot_general(x_ref[...], we_ref[...], (((1,), (1,)), ((), ())),
                        preferred_element_type=jnp.float32)
    h = jnp.maximum(h + be_ref[...], 0.0)
    hw = lax.dot_general(h, w0_ref[...], (((1,), (1,)), ((), ())),
                         preferred_element_type=jnp.float32)
    hp_ref[...] = hw * dv


def _pre_call(parts, x, we, be, w0):
    return pl.pallas_call(
        _pre_body,
        out_shape=[
            jax.ShapeDtypeStruct((_N, _D), jnp.float32),
            jax.ShapeDtypeStruct((_N, _D), jnp.float32),
        ],
    )(parts, x, we, be, w0)


def _layer_mm_body(p_ref, dv_ref, b_ref, r_ref, g_ref, be_ref, w_ref,
                   y_ref, hp_ref):
    y = (p_ref[0, :_N] + p_ref[1, :_N]) * dv_ref[...] + b_ref[...] + r_ref[...]
    y_ref[...] = y
    mu = jnp.mean(y, axis=0, keepdims=True)
    var = jnp.mean(y * y, axis=0, keepdims=True) - mu * mu
    h = jnp.maximum(
        (y - mu) * lax.rsqrt(var + _EPS) * g_ref[...] + be_ref[...], 0.0)
    hw = lax.dot_general(h, w_ref[...], (((1,), (1,)), ((), ())),
                         preferred_element_type=jnp.float32)
    hp_ref[...] = hw * dv_ref[...]


def _layer_mm0_body(p_ref, dv_ref, b_ref, g_ref, be_ref, w_ref, y_ref, hp_ref):
    y = (p_ref[0, :_N] + p_ref[1, :_N]) * dv_ref[...] + b_ref[...]
    y_ref[...] = y
    mu = jnp.mean(y, axis=0, keepdims=True)
    var = jnp.mean(y * y, axis=0, keepdims=True) - mu * mu
    h = jnp.maximum(
        (y - mu) * lax.rsqrt(var + _EPS) * g_ref[...] + be_ref[...], 0.0)
    hw = lax.dot_general(h, w_ref[...], (((1,), (1,)), ((), ())),
                         preferred_element_type=jnp.float32)
    hp_ref[...] = hw * dv_ref[...]


def _layer_call(parts, dinv, b, res, g, be, w_next):
    out_shape = [
        jax.ShapeDtypeStruct((_N, _D), jnp.float32),
        jax.ShapeDtypeStruct((_N, _D), jnp.float32),
    ]
    if res is None:
        return pl.pallas_call(_layer_mm0_body, out_shape=out_shape)(
            parts, dinv, b, g, be, w_next)
    return pl.pallas_call(_layer_mm_body, out_shape=out_shape)(
        parts, dinv, b, res, g, be, w_next)


def _layer_pred_body(p_ref, dv_ref, b_ref, r_ref, g_ref, be_ref, w_ref,
                     bp_ref, o_ref):
    y = (p_ref[0, :_N] + p_ref[1, :_N]) * dv_ref[...] + b_ref[...] + r_ref[...]
    mu = jnp.mean(y, axis=0, keepdims=True)
    var = jnp.mean(y * y, axis=0, keepdims=True) - mu * mu
    h = jnp.maximum(
        (y - mu) * lax.rsqrt(var + _EPS) * g_ref[...] + be_ref[...], 0.0)
    hw = lax.dot_general(h, w_ref[...], (((1,), (1,)), ((), ())),
                         preferred_element_type=jnp.float32)
    o_ref[...] = hw + bp_ref[...]


def _layer_pred_call(parts, dinv, b, res, g, be, wp, bp):
    return pl.pallas_call(
        _layer_pred_body,
        out_shape=jax.ShapeDtypeStruct((_N, _D), jnp.float32),
    )(parts, dinv, b, res, g, be, wp, bp)


def kernel(x, edge_index, W_enc, b_enc, Ws, bs, gammas, betas, W_pred, b_pred):
    roww = edge_index[0].reshape(_NW, _NCHUNK, _K)
    colw = edge_index[1].reshape(_NW, _NCHUNK, _K)

    dummy = jnp.zeros((_K, _D), jnp.float32)
    deg_parts = _sc_degree(colw, dummy)
    dinv, hp = _pre_call(deg_parts, x, W_enc, b_enc.reshape(1, _D), Ws[0])

    wp_pad = jnp.zeros((_D, _D), jnp.float32).at[:_C].set(W_pred)
    bp_pad = jnp.zeros((1, _D), jnp.float32).at[0, :_C].set(b_pred)

    y_last = None
    out = None
    for i in range(_L):
        parts = _sc_aggregate(hp, roww, colw, dummy)
        b = bs[i].reshape(1, _D)
        g = gammas[i].reshape(1, _D)
        be = betas[i].reshape(1, _D)
        if i < _L - 1:
            y, hp = _layer_call(parts, dinv, b, y_last, g, be, Ws[i + 1])
            y_last = y
        else:
            out = _layer_pred_call(parts, dinv, b, y_last, g, be, wp_pad,
                                   bp_pad)
    return out[:, :_C]
